# baseline (speedup 1.0000x reference)
import jax
import jax.numpy as jnp
from jax import lax
from jax.experimental import pallas as pl
from jax.experimental.pallas import tpu as pltpu
from jax.experimental.pallas import tpu_sc as plsc

NTOKEN = 1000000
EMB_DIM = 64
BATCH = 16384
HIST_LEN = 50

NC = 2
NS = 16
NW = NC * NS
LANES = 16

NPAIR = NTOKEN // 2
NBLK = (NTOKEN + 127) // 128
CB_PER_W = (BATCH // 128) // NW


def _iota16():
  return lax.iota(jnp.int32, LANES)


def _fmt_body(tt_hbm, tl_hbm, in_t, out_v, sem):
  wid = lax.axis_index("s") * NC + lax.axis_index("c")
  iot = _iota16()
  rows = [iot + LANES * h for h in range(4)]

  n_iter = (NBLK + NW - 1) // NW

  @pl.loop(0, n_iter)
  def _(k):
    c = wid + k * NW

    @pl.when(c < NBLK)
    def _():
      last = c == NBLK - 1
      col0 = c * 128
      for r in range(8):
        pltpu.async_copy(tt_hbm.at[pl.ds(8 * r, 8), pl.ds(col0, 128)],
                         in_t.at[pl.ds(8 * r, 8), :], sem)
      for r in range(8):
        pltpu.make_async_copy(tt_hbm.at[pl.ds(0, 8), pl.ds(0, 128)],
                              in_t.at[pl.ds(0, 8), :], sem).wait()
      for p in range(64):
        for h in range(8):
          cid = jnp.full((LANES,), 2 * p + (1 if h >= 4 else 0), jnp.int32)
          v = plsc.load_gather(in_t, [rows[h % 4], cid])
          out_v[p, pl.ds(LANES * h, LANES)] = v

      @pl.when(jnp.logical_not(last))
      def _():
        pltpu.sync_copy(out_v, tl_hbm.at[pl.ds(c * 64, 64), :])

      @pl.when(last)
      def _():
        pltpu.sync_copy(out_v.at[pl.ds(0, 32), :],
                        tl_hbm.at[pl.ds(NPAIR - 32, 32), :])


def _emb_pair_stage(xt_hbm, tl_hbm, xrow, idx2, par, gbuf, gsem):
  def go(t, cb):
    pltpu.sync_copy(xt_hbm.at[t, pl.ds(cb * 128, 128)], xrow)
    for h in range(8):
      v = xrow[pl.ds(LANES * h, LANES)]
      idx2[pl.ds(LANES * h, LANES)] = lax.shift_right_logical(v, 1)
      par[pl.ds(LANES * h, LANES)] = lax.bitwise_and(v, 1)
    pltpu.async_copy(tl_hbm.at[idx2], gbuf, gsem)
  return go


def _emb_body(tl_hbm, xt_hbm, ot_hbm, xrow0, idx20, par0, gbuf0, obuf0,
              xrow1, idx21, par1, gbuf1, obuf1, gsem0, gsem1, ssem0, ssem1):
  wid = lax.axis_index("s") * NC + lax.axis_index("c")
  cb0 = wid * CB_PER_W
  iot = _iota16()
  rows = [iot + LANES * h for h in range(8)]

  def wait_gather(gbuf, gsem):
    pltpu.make_async_copy(tl_hbm.at[idx20], gbuf, gsem).wait()

  def transpose(par, gbuf, obuf):
    for jg in range(8):
      parv = par[pl.ds(LANES * jg, LANES)]
      base = parv * EMB_DIM
      for d in range(EMB_DIM):
        v = plsc.load_gather(gbuf, [rows[jg], base + d])
        obuf[d, pl.ds(LANES * jg, LANES)] = v

  def store_start(t, cb, obuf, ssem):
    pltpu.async_copy(obuf, ot_hbm.at[t, :, pl.ds(cb * 128, 128)], ssem)

  def store_wait(obuf, ssem):
    pltpu.make_async_copy(obuf, ot_hbm.at[0, :, pl.ds(0, 128)], ssem).wait()

  stage0 = _emb_pair_stage(xt_hbm, tl_hbm, xrow0, idx20, par0, gbuf0, gsem0)
  stage1 = _emb_pair_stage(xt_hbm, tl_hbm, xrow1, idx21, par1, gbuf1, gsem1)

  @pl.loop(0, CB_PER_W)
  def _(cbi):
    cb = cb0 + cbi
    stage0(0, cb)

    @pl.loop(0, HIST_LEN // 2)
    def _(u):
      a = 2 * u
      stage1(a + 1, cb)
      wait_gather(gbuf0, gsem0)

      @pl.when(u > 0)
      def _():
        store_wait(obuf0, ssem0)
      transpose(par0, gbuf0, obuf0)
      store_start(a, cb, obuf0, ssem0)

      @pl.when(u < HIST_LEN // 2 - 1)
      def _():
        stage0(a + 2, cb)

      wait_gather(gbuf1, gsem1)

      @pl.when(u > 0)
      def _():
        store_wait(obuf1, ssem1)
      transpose(par1, gbuf1, obuf1)
      store_start(a + 1, cb, obuf1, ssem1)

    store_wait(obuf0, ssem0)
    store_wait(obuf1, ssem1)


@jax.jit
def _impl(x, emb_weight):
  mesh = plsc.VectorSubcoreMesh(
      core_axis_name="c", subcore_axis_name="s", num_cores=NC,
      num_subcores=NS)
  params = pltpu.CompilerParams(use_tc_tiling_on_sc=True,
                                needs_layout_passes=False)

  table_t = emb_weight.T
  x_t = x.astype(jnp.int32).T

  pair_table = pl.kernel(
      _fmt_body,
      out_type=jax.ShapeDtypeStruct((NPAIR, 128), jnp.float32),
      mesh=mesh,
      scratch_types=[
          pltpu.VMEM((EMB_DIM, 128), jnp.float32),
          pltpu.VMEM((EMB_DIM, 128), jnp.float32),
          pltpu.SemaphoreType.DMA,
      ],
      compiler_params=params)(table_t)

  out_t = pl.kernel(
      _emb_body,
      out_type=jax.ShapeDtypeStruct((HIST_LEN, EMB_DIM, BATCH), jnp.float32),
      mesh=mesh,
      scratch_types=[
          pltpu.VMEM((128,), jnp.int32),
          pltpu.VMEM((128,), jnp.int32),
          pltpu.VMEM((128,), jnp.int32),
          pltpu.VMEM((128, 128), jnp.float32),
          pltpu.VMEM((EMB_DIM, 128), jnp.float32),
          pltpu.VMEM((128,), jnp.int32),
          pltpu.VMEM((128,), jnp.int32),
          pltpu.VMEM((128,), jnp.int32),
          pltpu.VMEM((128, 128), jnp.float32),
          pltpu.VMEM((EMB_DIM, 128), jnp.float32),
          pltpu.SemaphoreType.DMA,
          pltpu.SemaphoreType.DMA,
          pltpu.SemaphoreType.DMA,
          pltpu.SemaphoreType.DMA,
      ],
      compiler_params=params)(pair_table, x_t)

  return jnp.transpose(out_t, (2, 0, 1))


def kernel(x, emb_weight):
  return _impl(x, emb_weight)


# speedup vs baseline: 1.2582x; 1.2582x over previous
import jax
import jax.numpy as jnp
from jax import lax
from jax.experimental import pallas as pl
from jax.experimental.pallas import tpu as pltpu
from jax.experimental.pallas import tpu_sc as plsc

NTOKEN = 1000000
EMB_DIM = 64
BATCH = 16384
HIST_LEN = 50

NC = 2
NS = 16
NW = NC * NS
LANES = 16

NPAIR = NTOKEN // 2
NBLK = (NTOKEN + 127) // 128
CB_PER_W = (BATCH // 128) // NW


def _iota16():
  return lax.iota(jnp.int32, LANES)


def _fmt_body(tt_hbm, tl_hbm, in_t, out_v, sem):
  wid = lax.axis_index("s") * NC + lax.axis_index("c")
  iot = _iota16()
  prow = [lax.shift_right_logical(iot + LANES * jg, 1) for jg in range(8)]
  pcol0 = lax.bitwise_and(iot, 1) * EMB_DIM

  n_iter = (NBLK + NW - 1) // NW

  @pl.loop(0, n_iter)
  def _(k):
    c = wid + k * NW

    @pl.when(c < NBLK)
    def _():
      last = c == NBLK - 1
      col0 = c * 128
      for r in range(8):
        pltpu.async_copy(tt_hbm.at[pl.ds(8 * r, 8), pl.ds(col0, 128)],
                         in_t.at[pl.ds(8 * r, 8), :], sem)
      for r in range(8):
        pltpu.make_async_copy(tt_hbm.at[pl.ds(0, 8), pl.ds(0, 128)],
                              in_t.at[pl.ds(0, 8), :], sem).wait()
      for d in range(EMB_DIM):
        pcol = pcol0 + d
        vs = [in_t[d, pl.ds(LANES * jg, LANES)] for jg in range(8)]
        for jg in range(8):
          plsc.store_scatter(out_v, [prow[jg], pcol], vs[jg])

      @pl.when(jnp.logical_not(last))
      def _():
        pltpu.sync_copy(out_v, tl_hbm.at[pl.ds(c * 64, 64), :])

      @pl.when(last)
      def _():
        pltpu.sync_copy(out_v.at[pl.ds(0, 32), :],
                        tl_hbm.at[pl.ds(NPAIR - 32, 32), :])


def _emb_pair_stage(xt_hbm, tl_hbm, xrow, idx2, par, gbuf, gsem):
  def go(t, cb):
    pltpu.sync_copy(xt_hbm.at[t, pl.ds(cb * 128, 128)], xrow)
    for h in range(8):
      v = xrow[pl.ds(LANES * h, LANES)]
      idx2[pl.ds(LANES * h, LANES)] = lax.shift_right_logical(v, 1)
      par[pl.ds(LANES * h, LANES)] = lax.bitwise_and(v, 1)
    pltpu.async_copy(tl_hbm.at[idx2], gbuf, gsem)
  return go


def _emb_body(tl_hbm, xt_hbm, ot_hbm, xrow0, idx20, par0, gbuf0, obuf0,
              xrow1, idx21, par1, gbuf1, obuf1, gsem0, gsem1, ssem0, ssem1):
  wid = lax.axis_index("s") * NC + lax.axis_index("c")
  cb0 = wid * CB_PER_W
  iot = _iota16()
  rows = [iot + LANES * h for h in range(8)]

  def wait_gather(gbuf, gsem):
    pltpu.make_async_copy(tl_hbm.at[idx20], gbuf, gsem).wait()

  def transpose(par, gbuf, obuf):
    base = [par[pl.ds(LANES * jg, LANES)] * EMB_DIM for jg in range(8)]
    for d in range(EMB_DIM):
      vs = [plsc.load_gather(gbuf, [rows[jg], base[jg] + d])
            for jg in range(8)]
      for jg in range(8):
        obuf[d, pl.ds(LANES * jg, LANES)] = vs[jg]

  def store_start(t, cb, obuf, ssem):
    pltpu.async_copy(obuf, ot_hbm.at[t, :, pl.ds(cb * 128, 128)], ssem)

  def store_wait(obuf, ssem):
    pltpu.make_async_copy(obuf, ot_hbm.at[0, :, pl.ds(0, 128)], ssem).wait()

  stage0 = _emb_pair_stage(xt_hbm, tl_hbm, xrow0, idx20, par0, gbuf0, gsem0)
  stage1 = _emb_pair_stage(xt_hbm, tl_hbm, xrow1, idx21, par1, gbuf1, gsem1)

  @pl.loop(0, CB_PER_W)
  def _(cbi):
    cb = cb0 + cbi
    stage0(0, cb)

    @pl.loop(0, HIST_LEN // 2)
    def _(u):
      a = 2 * u
      stage1(a + 1, cb)
      wait_gather(gbuf0, gsem0)

      @pl.when(u > 0)
      def _():
        store_wait(obuf0, ssem0)
      transpose(par0, gbuf0, obuf0)
      store_start(a, cb, obuf0, ssem0)

      @pl.when(u < HIST_LEN // 2 - 1)
      def _():
        stage0(a + 2, cb)

      wait_gather(gbuf1, gsem1)

      @pl.when(u > 0)
      def _():
        store_wait(obuf1, ssem1)
      transpose(par1, gbuf1, obuf1)
      store_start(a + 1, cb, obuf1, ssem1)

    store_wait(obuf0, ssem0)
    store_wait(obuf1, ssem1)


@jax.jit
def _impl(x, emb_weight):
  mesh = plsc.VectorSubcoreMesh(
      core_axis_name="c", subcore_axis_name="s", num_cores=NC,
      num_subcores=NS)
  params = pltpu.CompilerParams(use_tc_tiling_on_sc=True,
                                needs_layout_passes=False)

  table_t = emb_weight.T
  x_t = x.astype(jnp.int32).T

  pair_table = pl.kernel(
      _fmt_body,
      out_type=jax.ShapeDtypeStruct((NPAIR, 128), jnp.float32),
      mesh=mesh,
      scratch_types=[
          pltpu.VMEM((EMB_DIM, 128), jnp.float32),
          pltpu.VMEM((EMB_DIM, 128), jnp.float32),
          pltpu.SemaphoreType.DMA,
      ],
      compiler_params=params)(table_t)

  out_t = pl.kernel(
      _emb_body,
      out_type=jax.ShapeDtypeStruct((HIST_LEN, EMB_DIM, BATCH), jnp.float32),
      mesh=mesh,
      scratch_types=[
          pltpu.VMEM((128,), jnp.int32),
          pltpu.VMEM((128,), jnp.int32),
          pltpu.VMEM((128,), jnp.int32),
          pltpu.VMEM((128, 128), jnp.float32),
          pltpu.VMEM((EMB_DIM, 128), jnp.float32),
          pltpu.VMEM((128,), jnp.int32),
          pltpu.VMEM((128,), jnp.int32),
          pltpu.VMEM((128,), jnp.int32),
          pltpu.VMEM((128, 128), jnp.float32),
          pltpu.VMEM((EMB_DIM, 128), jnp.float32),
          pltpu.SemaphoreType.DMA,
          pltpu.SemaphoreType.DMA,
          pltpu.SemaphoreType.DMA,
          pltpu.SemaphoreType.DMA,
      ],
      compiler_params=params)(pair_table, x_t)

  return jnp.transpose(out_t, (2, 0, 1))


def kernel(x, emb_weight):
  return _impl(x, emb_weight)


# speedup vs baseline: 1.4669x; 1.1659x over previous
import jax
import jax.numpy as jnp
from jax import lax
from jax.experimental import pallas as pl
from jax.experimental.pallas import tpu as pltpu
from jax.experimental.pallas import tpu_sc as plsc

NTOKEN = 1000000
EMB_DIM = 64
BATCH = 16384
HIST_LEN = 50

NC = 2
NS = 16
NW = NC * NS
LANES = 16

NPAIR = NTOKEN // 2
NBLK = (NTOKEN + 127) // 128
CB_PER_W = (BATCH // 128) // NW


def _iota16():
  return lax.iota(jnp.int32, LANES)


def _fmt_body(tt_hbm, tl_hbm, in0, in1, out0, out1, isem0, isem1,
              osem0, osem1):
  wid = lax.axis_index("s") * NC + lax.axis_index("c")
  iot = _iota16()
  prow = [lax.shift_right_logical(iot + LANES * jg, 1) for jg in range(8)]
  pcol0 = lax.bitwise_and(iot, 1) * EMB_DIM

  n_iter = (NBLK + NW - 1) // NW + 1

  def fetch(c, in_t, isem):
    for r in range(8):
      pltpu.async_copy(tt_hbm.at[pl.ds(8 * r, 8), pl.ds(c * 128, 128)],
                       in_t.at[pl.ds(8 * r, 8), :], isem)

  def phase(k, in_cur, isem_cur, in_nxt, isem_nxt, out_cur, osem_cur):
    c = wid + k * NW

    @pl.when(c < NBLK)
    def _():
      c2 = c + NW

      @pl.when(c2 < NBLK)
      def _():
        fetch(c2, in_nxt, isem_nxt)
      for r in range(8):
        pltpu.make_async_copy(tt_hbm.at[pl.ds(0, 8), pl.ds(0, 128)],
                              in_cur.at[pl.ds(0, 8), :], isem_cur).wait()

      @pl.when(k >= 2)
      def _():
        pltpu.make_async_copy(out_cur, tl_hbm.at[pl.ds(0, 64), :],
                              osem_cur).wait()
      for d in range(EMB_DIM):
        pcol = pcol0 + d
        vs = [in_cur[d, pl.ds(LANES * jg, LANES)] for jg in range(8)]
        for jg in range(8):
          plsc.store_scatter(out_cur, [prow[jg], pcol], vs[jg])

      last = c == NBLK - 1

      @pl.when(jnp.logical_not(last))
      def _():
        pltpu.async_copy(out_cur, tl_hbm.at[pl.ds(c * 64, 64), :], osem_cur)

      @pl.when(last)
      def _():
        pltpu.async_copy(out_cur.at[pl.ds(0, 32), :],
                         tl_hbm.at[pl.ds(NPAIR - 32, 32), :], osem_cur)

  fetch(wid, in0, isem0)

  @pl.loop(0, n_iter // 2)
  def _(kk):
    phase(2 * kk, in0, isem0, in1, isem1, out0, osem0)
    phase(2 * kk + 1, in1, isem1, in0, isem0, out1, osem1)

  tail_wid = (NBLK - 1) % NW

  @pl.when(wid == tail_wid)
  def _():
    pltpu.make_async_copy(out0.at[pl.ds(0, 32), :],
                          tl_hbm.at[pl.ds(0, 32), :], osem0).wait()

  @pl.when(wid != tail_wid)
  def _():
    pltpu.make_async_copy(out0, tl_hbm.at[pl.ds(0, 64), :], osem0).wait()
  pltpu.make_async_copy(out1, tl_hbm.at[pl.ds(0, 64), :], osem1).wait()


def _emb_pair_stage(xall, tl_hbm, idx2, par, gbuf, gsem):
  def go(t):
    for h in range(8):
      v = xall[t, pl.ds(LANES * h, LANES)]
      idx2[pl.ds(LANES * h, LANES)] = lax.shift_right_logical(v, 1)
      par[pl.ds(LANES * h, LANES)] = lax.bitwise_and(v, 1)
    pltpu.async_copy(tl_hbm.at[idx2], gbuf, gsem)
  return go


def _emb_body(tl_hbm, xt_hbm, ot_hbm, xall, idx20, par0, gbuf0, obuf0,
              idx21, par1, gbuf1, obuf1, gsem0, gsem1, ssem0, ssem1):
  wid = lax.axis_index("s") * NC + lax.axis_index("c")
  cb0 = wid * CB_PER_W
  iot = _iota16()
  rows = [iot + LANES * h for h in range(8)]

  def wait_gather(gbuf, gsem):
    pltpu.make_async_copy(tl_hbm.at[idx20], gbuf, gsem).wait()

  def transpose(par, gbuf, obuf):
    base = [par[pl.ds(LANES * jg, LANES)] * EMB_DIM for jg in range(8)]
    for d in range(EMB_DIM):
      vs = [plsc.load_gather(gbuf, [rows[jg], base[jg] + d])
            for jg in range(8)]
      for jg in range(8):
        obuf[d, pl.ds(LANES * jg, LANES)] = vs[jg]

  def store_start(t, cb, obuf, ssem):
    pltpu.async_copy(obuf, ot_hbm.at[t, :, pl.ds(cb * 128, 128)], ssem)

  def store_wait(obuf, ssem):
    pltpu.make_async_copy(obuf, ot_hbm.at[0, :, pl.ds(0, 128)], ssem).wait()

  stage0 = _emb_pair_stage(xall, tl_hbm, idx20, par0, gbuf0, gsem0)
  stage1 = _emb_pair_stage(xall, tl_hbm, idx21, par1, gbuf1, gsem1)

  @pl.loop(0, CB_PER_W)
  def _(cbi):
    cb = cb0 + cbi
    pltpu.sync_copy(xt_hbm.at[pl.ds(0, HIST_LEN), pl.ds(cb * 128, 128)],
                    xall)
    stage0(0)

    @pl.loop(0, HIST_LEN // 2)
    def _(u):
      a = 2 * u
      stage1(a + 1)
      wait_gather(gbuf0, gsem0)

      @pl.when(u > 0)
      def _():
        store_wait(obuf0, ssem0)
      transpose(par0, gbuf0, obuf0)
      store_start(a, cb, obuf0, ssem0)

      @pl.when(u < HIST_LEN // 2 - 1)
      def _():
        stage0(a + 2)

      wait_gather(gbuf1, gsem1)

      @pl.when(u > 0)
      def _():
        store_wait(obuf1, ssem1)
      transpose(par1, gbuf1, obuf1)
      store_start(a + 1, cb, obuf1, ssem1)

    store_wait(obuf0, ssem0)
    store_wait(obuf1, ssem1)


@jax.jit
def _impl(x, emb_weight):
  mesh = plsc.VectorSubcoreMesh(
      core_axis_name="c", subcore_axis_name="s", num_cores=NC,
      num_subcores=NS)
  params = pltpu.CompilerParams(use_tc_tiling_on_sc=True,
                                needs_layout_passes=False)

  table_t = emb_weight.T
  x_t = x.astype(jnp.int32).T

  pair_table = pl.kernel(
      _fmt_body,
      out_type=jax.ShapeDtypeStruct((NPAIR, 128), jnp.float32),
      mesh=mesh,
      scratch_types=[
          pltpu.VMEM((EMB_DIM, 128), jnp.float32),
          pltpu.VMEM((EMB_DIM, 128), jnp.float32),
          pltpu.VMEM((EMB_DIM, 128), jnp.float32),
          pltpu.VMEM((EMB_DIM, 128), jnp.float32),
          pltpu.SemaphoreType.DMA,
          pltpu.SemaphoreType.DMA,
          pltpu.SemaphoreType.DMA,
          pltpu.SemaphoreType.DMA,
      ],
      compiler_params=params)(table_t)

  out_t = pl.kernel(
      _emb_body,
      out_type=jax.ShapeDtypeStruct((HIST_LEN, EMB_DIM, BATCH), jnp.float32),
      mesh=mesh,
      scratch_types=[
          pltpu.VMEM((HIST_LEN, 128), jnp.int32),
          pltpu.VMEM((128,), jnp.int32),
          pltpu.VMEM((128,), jnp.int32),
          pltpu.VMEM((128, 128), jnp.float32),
          pltpu.VMEM((EMB_DIM, 128), jnp.float32),
          pltpu.VMEM((128,), jnp.int32),
          pltpu.VMEM((128,), jnp.int32),
          pltpu.VMEM((128, 128), jnp.float32),
          pltpu.VMEM((EMB_DIM, 128), jnp.float32),
          pltpu.SemaphoreType.DMA,
          pltpu.SemaphoreType.DMA,
          pltpu.SemaphoreType.DMA,
          pltpu.SemaphoreType.DMA,
      ],
      compiler_params=params)(pair_table, x_t)

  return jnp.transpose(out_t, (2, 0, 1))


def kernel(x, emb_weight):
  return _impl(x, emb_weight)


# speedup vs baseline: 2.6289x; 1.7921x over previous
import jax
import jax.numpy as jnp
from jax import lax
from jax.experimental import pallas as pl
from jax.experimental.pallas import tpu as pltpu
from jax.experimental.pallas import tpu_sc as plsc

NTOKEN = 1000000
EMB_DIM = 64
BATCH = 16384
HIST_LEN = 50

NC = 2
NS = 16
NW = NC * NS

NB = BATCH // NW
G = 16
N_GROUPS = NB // G
T = N_GROUPS // 2


def _body(table_hbm, x_hbm, out_hbm, slab, buf0, buf1, gsem0, gsem1,
          ssem0, ssem1):
  wid = lax.axis_index("s") * NC + lax.axis_index("c")
  b0 = wid * NB

  pltpu.sync_copy(x_hbm.at[pl.ds(b0, NB), :], slab)

  def issue_gathers(g, buf, sem):
    for i in range(G):
      pltpu.async_copy(table_hbm.at[slab.at[g * G + i]], buf.at[i], sem)

  def wait_gathers(buf, sem):
    for i in range(G):
      pltpu.make_async_copy(table_hbm.at[slab.at[i]], buf.at[i], sem).wait()

  def issue_store(g, buf, sem):
    pltpu.async_copy(buf, out_hbm.at[pl.ds(b0 + g * G, G), :, :], sem)

  def wait_store(buf, sem):
    pltpu.make_async_copy(buf, out_hbm.at[pl.ds(b0, G), :, :], sem).wait()

  issue_gathers(0, buf0, gsem0)

  @pl.loop(0, T)
  def _(t):
    a = 2 * t

    @pl.when(t > 0)
    def _():
      wait_store(buf1, ssem1)
    issue_gathers(a + 1, buf1, gsem1)

    wait_gathers(buf0, gsem0)
    issue_store(a, buf0, ssem0)

    @pl.when(t < T - 1)
    def _():
      wait_store(buf0, ssem0)
      issue_gathers(a + 2, buf0, gsem0)

    wait_gathers(buf1, gsem1)
    issue_store(a + 1, buf1, ssem1)

  wait_store(buf0, ssem0)
  wait_store(buf1, ssem1)


@jax.jit
def _lookup(x2d, emb_weight):
  mesh = plsc.VectorSubcoreMesh(
      core_axis_name="c", subcore_axis_name="s", num_cores=NC,
      num_subcores=NS)
  scratch = [
      pltpu.VMEM((NB, HIST_LEN), jnp.int32),
      pltpu.VMEM((G, HIST_LEN, EMB_DIM), jnp.float32),
      pltpu.VMEM((G, HIST_LEN, EMB_DIM), jnp.float32),
      pltpu.SemaphoreType.DMA,
      pltpu.SemaphoreType.DMA,
      pltpu.SemaphoreType.DMA,
      pltpu.SemaphoreType.DMA,
  ]
  return pl.kernel(
      _body,
      out_type=jax.ShapeDtypeStruct((BATCH, HIST_LEN, EMB_DIM), jnp.float32),
      mesh=mesh,
      scratch_types=scratch,
      compiler_params=pltpu.CompilerParams(use_tc_tiling_on_sc=False),
  )(emb_weight, x2d)


def kernel(x, emb_weight):
  return _lookup(x.astype(jnp.int32), emb_weight)
